# baseline (device time: 42258 ns/iter reference)
import jax
import jax.numpy as jnp
from jax import lax
from jax.experimental import pallas as pl
from jax.experimental.pallas import tpu as pltpu

N_CHUNKS = 4


def kernel(O, Wo):
    B, S, H, D = O.shape
    K = H * D
    N = Wo.shape[1]
    Sh = S // 2
    C = Sh // N_CHUNKS
    R = B * C + 1

    O2T = jnp.swapaxes(O.reshape(B, S, K), 1, 2)

    t_dims = (((0,), (0,)), ((), ()))

    def body(
        o_hbm,
        w_ref,
        out_ref,
        o_vmem,
        acc_ref,
        send_buf,
        recv_buf,
        in_sems,
        send_sems,
        recv_sems,
        out_sems,
    ):
        my_x = lax.axis_index("x")
        my_y = lax.axis_index("y")
        my_z = lax.axis_index("z")
        other = 1 - my_x
        partner = (other, my_y, my_z)

        part_start = other * Sh
        my_start = my_x * Sh

        in_copies = []
        for c in range(N_CHUNKS):
            for b in range(B):
                cp = pltpu.make_async_copy(
                    o_hbm.at[b, :, pl.ds(part_start + c * C, C)],
                    o_vmem.at[b, :, pl.ds(part_start + c * C, C)],
                    in_sems.at[c * B + b],
                )
                cp.start()
                in_copies.append(cp)
        own_copies = []
        for b in range(B):
            cp = pltpu.make_async_copy(
                o_hbm.at[b, :, pl.ds(my_start, Sh)],
                o_vmem.at[b, :, pl.ds(my_start, Sh)],
                in_sems.at[N_CHUNKS * B + b],
            )
            cp.start()
            own_copies.append(cp)

        barrier_sem = pltpu.get_barrier_semaphore()
        pl.semaphore_signal(
            barrier_sem,
            inc=1,
            device_id=partner,
            device_id_type=pl.DeviceIdType.MESH,
        )
        pl.semaphore_wait(barrier_sem, 1)

        rdmas = []
        for c in range(N_CHUNKS):
            ps = []
            for b in range(B):
                in_copies[c * B + b].wait()
                ps.append(
                    lax.dot_general(
                        o_vmem[b, :, pl.ds(part_start + c * C, C)],
                        w_ref[...],
                        t_dims,
                        preferred_element_type=jnp.float32,
                    )
                )
            amax = jnp.maximum(
                jnp.max(jnp.abs(ps[0])), jnp.max(jnp.abs(ps[1]))
            )
            e = jnp.ceil(jnp.log2(jnp.maximum(amax, 1e-20)))
            qscale = 127.0 * jnp.exp2(-e)
            for b in range(B):
                send_buf[c, pl.ds(b * C, C), :] = jnp.round(
                    ps[b] * qscale
                ).astype(jnp.int8)
            send_buf[c, pl.ds(B * C, 1), :] = jnp.full(
                (1, N), e, jnp.float32
            ).astype(jnp.int8)
            rdma = pltpu.make_async_remote_copy(
                src_ref=send_buf.at[c],
                dst_ref=recv_buf.at[c],
                send_sem=send_sems.at[c],
                recv_sem=recv_sems.at[c],
                device_id=partner,
                device_id_type=pl.DeviceIdType.MESH,
            )
            rdma.start()
            rdmas.append(rdma)

        for b in range(B):
            own_copies[b].wait()
            acc_ref[b] = lax.dot_general(
                o_vmem[b, :, pl.ds(my_start, Sh)],
                w_ref[...],
                t_dims,
                preferred_element_type=jnp.float32,
            )

        out_copies = []
        for c in range(N_CHUNKS):
            rdmas[c].wait_recv()
            e_row = recv_buf[c, pl.ds(B * C, 1), :].astype(jnp.float32)
            factor = jnp.exp2(e_row) * (1.0 / 127.0)
            for b in range(B):
                q = recv_buf[c, pl.ds(b * C, C), :].astype(jnp.float32)
                acc_ref[b, pl.ds(c * C, C)] += q * factor
                cp = pltpu.make_async_copy(
                    acc_ref.at[b, pl.ds(c * C, C)],
                    out_ref.at[b, pl.ds(c * C, C)],
                    out_sems.at[c * B + b],
                )
                cp.start()
                out_copies.append(cp)

        for cp in out_copies:
            cp.wait()
        for rdma in rdmas:
            rdma.wait_send()

    return pl.pallas_call(
        body,
        out_shape=jax.ShapeDtypeStruct((B, Sh, N), jnp.float32),
        in_specs=[
            pl.BlockSpec(memory_space=pltpu.MemorySpace.HBM),
            pl.BlockSpec(memory_space=pltpu.VMEM),
        ],
        out_specs=pl.BlockSpec(memory_space=pltpu.MemorySpace.HBM),
        scratch_shapes=[
            pltpu.VMEM((B, K, S), jnp.float32),
            pltpu.VMEM((B, Sh, N), jnp.float32),
            pltpu.VMEM((N_CHUNKS, R, N), jnp.int8),
            pltpu.VMEM((N_CHUNKS, R, N), jnp.int8),
            pltpu.SemaphoreType.DMA((N_CHUNKS * B + B,)),
            pltpu.SemaphoreType.DMA((N_CHUNKS,)),
            pltpu.SemaphoreType.DMA((N_CHUNKS,)),
            pltpu.SemaphoreType.DMA((N_CHUNKS * B,)),
        ],
        compiler_params=pltpu.CompilerParams(collective_id=0),
    )(O2T, Wo)


# device time: 39763 ns/iter; 1.0627x vs baseline; 1.0627x over previous
import jax
import jax.numpy as jnp
from jax import lax
from jax.experimental import pallas as pl
from jax.experimental.pallas import tpu as pltpu

N_CHUNKS = 4


def kernel(O, Wo):
    B, S, H, D = O.shape
    K = H * D
    N = Wo.shape[1]
    Sh = S // 2
    C = Sh // N_CHUNKS
    R = B * C + 1

    O2T = jnp.swapaxes(O.reshape(B, S, K), 1, 2)

    t_dims = (((0,), (0,)), ((), ()))

    def body(
        o_ref,
        w_ref,
        out_ref,
        acc_ref,
        send_buf,
        recv_buf,
        send_sems,
        recv_sems,
        out_sems,
    ):
        my_x = lax.axis_index("x")
        my_y = lax.axis_index("y")
        my_z = lax.axis_index("z")
        other = 1 - my_x
        partner = (other, my_y, my_z)

        barrier_sem = pltpu.get_barrier_semaphore()
        pl.semaphore_signal(
            barrier_sem,
            inc=1,
            device_id=partner,
            device_id_type=pl.DeviceIdType.MESH,
        )
        pl.semaphore_wait(barrier_sem, 1)

        part_start = other * Sh
        rdmas = []
        for c in range(N_CHUNKS):
            ps = []
            for b in range(B):
                ps.append(
                    lax.dot_general(
                        o_ref[b, :, pl.ds(part_start + c * C, C)],
                        w_ref[...],
                        t_dims,
                        preferred_element_type=jnp.float32,
                    )
                )
            amax = jnp.maximum(
                jnp.max(jnp.abs(ps[0])), jnp.max(jnp.abs(ps[1]))
            )
            e = jnp.ceil(jnp.log2(jnp.maximum(amax, 1e-20)))
            qscale = 127.0 * jnp.exp2(-e)
            for b in range(B):
                send_buf[c, pl.ds(b * C, C), :] = jnp.round(
                    ps[b] * qscale
                ).astype(jnp.int8)
            send_buf[c, pl.ds(B * C, 1), :] = jnp.full(
                (1, N), e, jnp.float32
            ).astype(jnp.int8)
            rdma = pltpu.make_async_remote_copy(
                src_ref=send_buf.at[c],
                dst_ref=recv_buf.at[c],
                send_sem=send_sems.at[c],
                recv_sem=recv_sems.at[c],
                device_id=partner,
                device_id_type=pl.DeviceIdType.MESH,
            )
            rdma.start()
            rdmas.append(rdma)

        my_start = my_x * Sh
        for b in range(B):
            acc_ref[b] = lax.dot_general(
                o_ref[b, :, pl.ds(my_start, Sh)],
                w_ref[...],
                t_dims,
                preferred_element_type=jnp.float32,
            )

        out_copies = []
        for c in range(N_CHUNKS):
            rdmas[c].wait_recv()
            e_row = recv_buf[c, pl.ds(B * C, 1), :].astype(jnp.float32)
            factor = jnp.exp2(e_row) * (1.0 / 127.0)
            for b in range(B):
                q = recv_buf[c, pl.ds(b * C, C), :].astype(jnp.float32)
                acc_ref[b, pl.ds(c * C, C)] += q * factor
                cp = pltpu.make_async_copy(
                    acc_ref.at[b, pl.ds(c * C, C)],
                    out_ref.at[b, pl.ds(c * C, C)],
                    out_sems.at[c * B + b],
                )
                cp.start()
                out_copies.append(cp)

        for cp in out_copies:
            cp.wait()
        for rdma in rdmas:
            rdma.wait_send()

    return pl.pallas_call(
        body,
        out_shape=jax.ShapeDtypeStruct((B, Sh, N), jnp.float32),
        in_specs=[
            pl.BlockSpec(memory_space=pltpu.VMEM),
            pl.BlockSpec(memory_space=pltpu.VMEM),
        ],
        out_specs=pl.BlockSpec(memory_space=pl.ANY),
        scratch_shapes=[
            pltpu.VMEM((B, Sh, N), jnp.float32),
            pltpu.VMEM((N_CHUNKS, R, N), jnp.int8),
            pltpu.VMEM((N_CHUNKS, R, N), jnp.int8),
            pltpu.SemaphoreType.DMA((N_CHUNKS,)),
            pltpu.SemaphoreType.DMA((N_CHUNKS,)),
            pltpu.SemaphoreType.DMA((N_CHUNKS * B,)),
        ],
        compiler_params=pltpu.CompilerParams(collective_id=0),
    )(O2T, Wo)
